# baseline (speedup 1.0000x reference)
import jax
import jax.numpy as jnp
from jax import lax
from jax.experimental import pallas as pl
from jax.experimental.pallas import tpu as pltpu
from jax.experimental.pallas import tpu_sc as plsc

import functools


def _make_sc_kernel(S, W):
    info = plsc.get_sparse_core_info()
    NC, NS, L = info.num_cores, info.num_subcores, info.num_lanes
    NW = NC * NS
    mid = S // 2
    assert mid % NW == 0
    K = mid // NW
    assert K % L == 0

    mesh = plsc.VectorSubcoreMesh(core_axis_name="c", subcore_axis_name="s")

    @functools.partial(
        pl.kernel,
        out_type=jax.ShapeDtypeStruct((S, W), jnp.float32),
        mesh=mesh,
        scratch_types=[
            pltpu.VMEM((K, W), jnp.float32),
            pltpu.VMEM((K,), jnp.int32),
            pltpu.SemaphoreType.DMA,
        ],
    )
    def k(emb_hbm, out_hbm, buf, idx, sem):
        wid = lax.axis_index("s") * NC + lax.axis_index("c")
        d0 = wid * K

        pltpu.sync_copy(emb_hbm.at[pl.ds(d0, K)], buf)

        for c in range(K // L):
            base = mid - d0 - c * L
            idx[pl.ds(c * L, L)] = base - lax.iota(jnp.int32, L)

        c1 = pltpu.async_copy(buf, out_hbm.at[pl.ds(mid + d0, K)], sem)
        c2 = pltpu.async_copy(buf, out_hbm.at[idx], sem)
        c1.wait()
        c2.wait()

        @pl.when(wid == NW - 1)
        def _():
            pltpu.sync_copy(emb_hbm.at[pl.ds(mid, 1)], buf.at[pl.ds(0, 1)])
            pltpu.sync_copy(buf.at[pl.ds(0, 1)], out_hbm.at[pl.ds(0, 1)])

    return k


def _make_tc_kernel(NB, S, W, BS=256):
    mid = S // 2
    assert mid % BS == 0
    nd = mid // BS
    grid = (2 * nd + 1,)

    def emb_index(j):
        return (jnp.where(j <= nd, nd - j, j - nd - 1), 0)

    def out_index(j):
        return (0, jnp.maximum(j - 1, 0), 0)

    def body(emb_ref, out_ref, carry):
        j = pl.program_id(0)
        block = emb_ref[...]

        @pl.when(jnp.logical_and(j >= 1, j <= nd))
        def _():
            ii = lax.broadcasted_iota(jnp.int32, (BS, BS), 0)
            jj = lax.broadcasted_iota(jnp.int32, (BS, BS), 1)
            anti = (ii + jj == BS).astype(jnp.float32)
            shifted = jax.lax.dot(anti, block, preferred_element_type=jnp.float32)
            si = lax.broadcasted_iota(jnp.int32, (BS, 8), 0)
            sj = lax.broadcasted_iota(jnp.int32, (BS, 8), 1)
            sel = jnp.logical_and(si == 0, sj == 0).astype(jnp.float32)
            desc = shifted + jax.lax.dot(sel, carry[...], preferred_element_type=jnp.float32)
            out_ref[...] = jnp.broadcast_to(desc[None], (NB, BS, W))

        @pl.when(j > nd)
        def _():
            out_ref[...] = jnp.broadcast_to(block[None], (NB, BS, W))

        carry[...] = block[0:8]

    return pl.pallas_call(
        body,
        grid=grid,
        in_specs=[pl.BlockSpec((BS, W), emb_index)],
        out_specs=pl.BlockSpec((NB, BS, W), out_index),
        out_shape=jax.ShapeDtypeStruct((NB, S, W), jnp.float32),
        scratch_shapes=[pltpu.VMEM((8, W), jnp.float32)],
    )


def kernel(inputs, embeddings):
    B, S, W = inputs.shape
    sc_out = _make_sc_kernel(S, W)(embeddings).reshape(1, S, W)
    tc_out = _make_tc_kernel(B - 1, S, W)(embeddings)
    return jnp.concatenate([sc_out, tc_out], axis=0)


# speedup vs baseline: 2.0928x; 2.0928x over previous
import jax
import jax.numpy as jnp
from jax import lax
from jax.experimental import pallas as pl
from jax.experimental.pallas import tpu as pltpu
from jax.experimental.pallas import tpu_sc as plsc

import functools


def _make_sc_kernel(B, S, W):
    info = plsc.get_sparse_core_info()
    NC, NS, L = info.num_cores, info.num_subcores, info.num_lanes
    NW = NC * NS
    mid = S // 2
    assert mid % NW == 0
    K = mid // NW
    assert K % L == 0

    mesh = plsc.VectorSubcoreMesh(core_axis_name="c", subcore_axis_name="s")

    @functools.partial(
        pl.kernel,
        out_type=jax.ShapeDtypeStruct((B * S, W), jnp.float32),
        mesh=mesh,
        scratch_types=[
            pltpu.VMEM((K, W), jnp.float32),
            pltpu.VMEM((B, K), jnp.int32),
            pltpu.VMEM((1, W), jnp.float32),
            pltpu.SemaphoreType.DMA,
            pltpu.SemaphoreType.DMA,
        ],
    )
    def k(emb_hbm, out_hbm, buf, idx, x0, sem, gsem):
        wid = lax.axis_index("s") * NC + lax.axis_index("c")
        d0 = wid * K
        last = wid == NW - 1

        pltpu.async_copy(emb_hbm.at[pl.ds(d0, K)], buf, gsem)

        @pl.when(last)
        def _():
            pltpu.async_copy(emb_hbm.at[pl.ds(mid, 1)], x0, gsem)

        for b in range(B):
            for c in range(K // L):
                base = b * S + mid - d0 - c * L
                idx[b, pl.ds(c * L, L)] = base - lax.iota(jnp.int32, L)

        pltpu.make_async_copy(emb_hbm.at[pl.ds(d0, K)], buf, gsem).wait()

        copies = []
        for b in range(B):
            copies.append(
                pltpu.async_copy(buf, out_hbm.at[pl.ds(b * S + mid + d0, K)], sem)
            )
            copies.append(pltpu.async_copy(buf, out_hbm.at[idx.at[b]], sem))

        @pl.when(last)
        def _():
            pltpu.make_async_copy(emb_hbm.at[pl.ds(mid, 1)], x0, gsem).wait()
            for b in range(B):
                pltpu.async_copy(x0, out_hbm.at[pl.ds(b * S, 1)], sem).wait()

        for cp in copies:
            cp.wait()

    return k


def kernel(inputs, embeddings):
    B, S, W = inputs.shape
    out = _make_sc_kernel(B, S, W)(embeddings)
    return out.reshape(B, S, W)
